# baseline (speedup 1.0000x reference)
import functools

import jax
import jax.numpy as jnp
from jax import lax
from jax.experimental import pallas as pl
from jax.experimental.pallas import tpu as pltpu
from jax.experimental.pallas import tpu_sc as plsc

ROWS, COLS = 4096, 200
D = 64
SCALE = 8.0
NC, NS = 2, 16
NW = NC * NS
B = ROWS * COLS
K = 128
BPW = B // NW
NCHUNK = BPW // K
NBUF = 4
L = 16


def _emb_body(idx_hbm, table_hbm, out_hbm, idx_v, rows_v, obuf_v, gsem, osem):
    wid = lax.axis_index("s") * NC + lax.axis_index("c")
    base = wid * BPW

    pltpu.sync_copy(idx_hbm.at[wid], idx_v)

    for b in range(NBUF):
        pltpu.async_copy(table_hbm.at[idx_v.at[b]], rows_v.at[b], gsem.at[b])

    def outer(g, carry):
        for b in range(NBUF):
            c = g * NBUF + b
            pltpu.make_async_copy(
                table_hbm.at[idx_v.at[0]], rows_v.at[b], gsem.at[b]
            ).wait()

            @pl.when(g > 0)
            def _():
                pltpu.make_async_copy(
                    obuf_v.at[b], out_hbm.at[pl.ds(0, K)], osem.at[b]
                ).wait()

            def scale_row(r, _):
                for q in range(D // L):
                    obuf_v[b, r, pl.ds(q * L, L)] = (
                        rows_v[b, r, pl.ds(q * L, L)] * SCALE
                    )
                return 0

            lax.fori_loop(0, K, scale_row, 0, unroll=2)

            pltpu.async_copy(
                obuf_v.at[b], out_hbm.at[pl.ds(base + c * K, K)], osem.at[b]
            )

            cn = c + NBUF

            @pl.when(cn < NCHUNK)
            def _():
                pltpu.async_copy(
                    table_hbm.at[idx_v.at[cn]], rows_v.at[b], gsem.at[b]
                )

        return carry

    lax.fori_loop(0, NCHUNK // NBUF, outer, 0)

    for b in range(NBUF):
        pltpu.make_async_copy(
            obuf_v.at[b], out_hbm.at[pl.ds(0, K)], osem.at[b]
        ).wait()


@jax.jit
def _embed(idx, table):
    mesh = plsc.VectorSubcoreMesh(
        core_axis_name="c", subcore_axis_name="s", num_cores=NC, num_subcores=NS
    )
    fn = pl.kernel(
        _emb_body,
        out_type=jax.ShapeDtypeStruct((B, D), jnp.float32),
        mesh=mesh,
        compiler_params=pltpu.CompilerParams(use_tc_tiling_on_sc=False),
        scratch_types=[
            pltpu.VMEM((NCHUNK, K), jnp.int32),
            pltpu.VMEM((NBUF, K, D), jnp.float32),
            pltpu.VMEM((NBUF, K, D), jnp.float32),
            pltpu.SemaphoreType.DMA((NBUF,)),
            pltpu.SemaphoreType.DMA((NBUF,)),
        ],
    )
    return fn(idx, table)


def kernel(tokens, table):
    idx = tokens.reshape(NW, NCHUNK, K)
    out = _embed(idx, table)
    return out.reshape(ROWS, COLS, D)
